# baseline (speedup 1.0000x reference)
import jax
import jax.numpy as jnp
import numpy as np
from jax import lax
from jax.experimental import pallas as pl
from jax.experimental.pallas import tpu as pltpu
from jax.experimental.pallas import tpu_sc as plsc

_N = 32768
_K = 8192
_L = 16
_NW = 16
_CHUNK = _N // _NW
_NV = _CHUNK // _L
_NB = 256
_NBV = _NB // _L
_FOFF = 4 * _NB

_MININT = np.int32(-(2**31))
_M31 = np.int32(0x7FFFFFFF)


def _srl(x, amount):
    return lax.shift_right_logical(x, jnp.full(x.shape, amount, jnp.int32))


def _sc_body(x_hbm, out_hbm, xv, ub, hist, hist2, gh, bi3, bi2, bi1, bi0,
             fidx, finv, faccv, outv, shacc, shaccf, dsem):
    sid = lax.axis_index("s")
    cid = lax.axis_index("c")
    base = sid * _CHUNK
    xcopy = pltpu.async_copy(x_hbm.at[pl.ds(base, _CHUNK)], xv, dsem)

    ones = jnp.ones((_L,), jnp.int32)
    lane = jnp.arange(_L, dtype=jnp.int32)

    def ib(v, c):
        idx = lane + v * _L
        bi3[pl.ds(v * _L, _L)] = idx
        bi2[pl.ds(v * _L, _L)] = idx + _NB
        bi1[pl.ds(v * _L, _L)] = idx + 2 * _NB
        bi0[pl.ds(v * _L, _L)] = idx + 3 * _NB
        gh[pl.ds(v * _L, _L)] = jnp.zeros((_L,), jnp.int32)
        return c

    lax.fori_loop(0, _NBV, ib, jnp.int32(0))
    fidx[...] = lane

    @pl.when(sid < 4)
    def _():
        pltpu.sync_copy(gh, shacc.at[pl.ds(sid * _NB, _NB)])

    @pl.when(sid == 4)
    def _():
        finv[...] = jnp.zeros((_L,), jnp.float32)
        pltpu.sync_copy(finv, shaccf)

    with jax.named_scope("xload"):
        xcopy.wait()

    phi = jnp.int32(0)
    kr = jnp.int32(_K)

    for p in (3, 2, 1, 0):
        shift = 8 * p

        def zb(v, c):
            hist[pl.ds(v * _L, _L)] = jnp.zeros((_L,), jnp.int32)
            hist2[pl.ds(v * _L, _L)] = jnp.zeros((_L,), jnp.int32)
            return c

        lax.fori_loop(0, _NBV, zb, jnp.int32(0))

        if p == 3:
            def hb(i, c):
                for j, h in ((0, hist), (1, hist2)):
                    off = i * (2 * _L) + j * _L
                    xb = xv[pl.ds(off, _L)]
                    ibits = plsc.bitcast(xb, jnp.int32)
                    ks = jnp.where(ibits >= 0, ibits, ibits ^ _M31)
                    u = ks ^ _MININT
                    ub[pl.ds(off, _L)] = u
                    f = _srl(u, 24) & 0xFF
                    plsc.addupdate_scatter(h, [f], ones)
                return c
        else:
            phiv = jnp.full((_L,), phi, jnp.int32)

            def hb(i, c, _shift=shift, _phiv=phiv):
                for j, h in ((0, hist), (1, hist2)):
                    off = i * (2 * _L) + j * _L
                    u = ub[pl.ds(off, _L)]
                    act = _srl(u, _shift + 8) == _phiv
                    f = _srl(u, _shift) & 0xFF
                    plsc.addupdate_scatter(h, [f], ones, mask=act)
                return c

        with jax.named_scope(f"histsweep{p}"):
            lax.fori_loop(0, _NV // 2, hb, jnp.int32(0))

        def cmb(v, c):
            hist[pl.ds(v * _L, _L)] = (hist[pl.ds(v * _L, _L)] +
                                       hist2[pl.ds(v * _L, _L)])
            return c

        lax.fori_loop(0, _NBV, cmb, jnp.int32(0))

        bip = {3: bi3, 2: bi2, 1: bi1, 0: bi0}[p]
        with jax.named_scope(f"comm{p}"):
            if p == 3:
                plsc.subcore_barrier()
            pltpu.sync_copy(hist, shacc.at[bip], add=True)
            plsc.subcore_barrier()
            pltpu.sync_copy(shacc.at[pl.ds((3 - p) * _NB, _NB)], gh)

        with jax.named_scope(f"scan{p}"):
            bidx = lane * _L
            tvec = plsc.load_gather(gh, [bidx])
            for j in range(1, _L):
                tvec = tvec + plsc.load_gather(gh, [bidx + j])
            st = lax.rev(plsc.cumsum(lax.rev(tvec, (0,))), (0,))
            condv = st >= kr
            vstar = jnp.sum(jnp.where(condv, 1, 0)) - 1
            stv = jnp.min(jnp.where(condv, st, _M31))
            hv = gh[pl.ds(vstar * _L, _L)]
            tv_ = jnp.sum(hv)
            s2 = lax.rev(plsc.cumsum(lax.rev(hv, (0,))), (0,)) + (stv - tv_)
            cond2 = s2 >= kr
            c2 = jnp.sum(jnp.where(cond2, 1, 0))
            sel_in = jnp.sum(jnp.where(cond2, hv, 0))
            bstar = vstar * _L + c2 - 1
            kr = kr - stv + sel_in
            phi = lax.shift_left(phi, jnp.int32(8)) | bstar

    phis = phi ^ _MININT

    def sm(i, a):
        for j in range(4):
            off = i * (4 * _L) + j * _L
            u = ub[pl.ds(off, _L)]
            xb = xv[pl.ds(off, _L)]
            gt = (u ^ _MININT) > phis
            a = a + jnp.where(gt, xb, jnp.float32(0))
        return a

    with jax.named_scope("finalsweep"):
        acc = lax.fori_loop(0, _NV // 4, sm, jnp.zeros((_L,), jnp.float32))
    sloc = jnp.sum(acc)

    with jax.named_scope("fpub"):
        finv[...] = jnp.full((_L,), sloc, jnp.float32)
        pltpu.sync_copy(finv, shaccf.at[fidx], add=True)
        plsc.subcore_barrier()
        pltpu.sync_copy(shaccf, faccv)

    stot = faccv[...]

    pv = jnp.full((_L,), phi, jnp.int32)
    ksv = pv ^ _MININT
    bits = jnp.where(ksv >= 0, ksv, ksv ^ _M31)
    tv = plsc.bitcast(bits, jnp.float32)
    krf = jnp.full((_L,), kr, jnp.int32).astype(jnp.float32)
    res = (stot + krf * tv) * jnp.float32(1.0 / _K)

    @pl.when((cid == 0) & (sid == 0))
    def _():
        outv[...] = res
        pltpu.sync_copy(outv.at[pl.ds(0, 1)], out_hbm)


def _make_kernel():
    mesh = plsc.VectorSubcoreMesh(core_axis_name="c", subcore_axis_name="s",
                                  num_cores=1)
    return pl.kernel(
        _sc_body,
        out_type=jax.ShapeDtypeStruct((1,), jnp.float32),
        mesh=mesh,
        compiler_params=pltpu.CompilerParams(needs_layout_passes=False),
        scratch_types=[
            pltpu.VMEM((_CHUNK,), jnp.float32),
            pltpu.VMEM((_CHUNK,), jnp.int32),
            pltpu.VMEM((_NB,), jnp.int32),
            pltpu.VMEM((_NB,), jnp.int32),
            pltpu.VMEM((_NB,), jnp.int32),
            pltpu.VMEM((_NB,), jnp.int32),
            pltpu.VMEM((_NB,), jnp.int32),
            pltpu.VMEM((_NB,), jnp.int32),
            pltpu.VMEM((_NB,), jnp.int32),
            pltpu.VMEM((_L,), jnp.int32),
            pltpu.VMEM((_L,), jnp.float32),
            pltpu.VMEM((_L,), jnp.float32),
            pltpu.VMEM((_L,), jnp.float32),
            pltpu.VMEM_SHARED((_FOFF,), jnp.int32),
            pltpu.VMEM_SHARED((_L,), jnp.float32),
            pltpu.SemaphoreType.DMA,
        ],
    )


def kernel(x):
    out = _make_kernel()(x)
    return out.reshape(())


# speedup vs baseline: 1.0206x; 1.0206x over previous
import jax
import jax.numpy as jnp
import numpy as np
from jax import lax
from jax.experimental import pallas as pl
from jax.experimental.pallas import tpu as pltpu
from jax.experimental.pallas import tpu_sc as plsc

_N = 32768
_K = 8192
_L = 16
_NW = 16
_CHUNK = _N // _NW
_NV = _CHUNK // _L
_NB = 256
_NBV = _NB // _L
_FOFF = 4 * _NB

_MININT = np.int32(-(2**31))
_M31 = np.int32(0x7FFFFFFF)


def _srl(x, amount):
    return lax.shift_right_logical(x, jnp.full(x.shape, amount, jnp.int32))


_HALF = _CHUNK // 2


def _sc_body(x_hbm, out_hbm, xv, ub, hist, gh, bi3, bi2, bi1, bi0,
             fidx, finv, faccv, outv, shacc, shaccf, dsem, dsem2):
    sid = lax.axis_index("s")
    cid = lax.axis_index("c")
    base = sid * _CHUNK
    xcopy = pltpu.async_copy(x_hbm.at[pl.ds(base, _HALF)],
                             xv.at[pl.ds(0, _HALF)], dsem)
    xcopy2 = pltpu.async_copy(x_hbm.at[pl.ds(base + _HALF, _HALF)],
                              xv.at[pl.ds(_HALF, _HALF)], dsem2)

    ones = jnp.ones((_L,), jnp.int32)
    lane = jnp.arange(_L, dtype=jnp.int32)

    def ib(v, c):
        idx = lane + v * _L
        bi3[pl.ds(v * _L, _L)] = idx
        bi2[pl.ds(v * _L, _L)] = idx + _NB
        bi1[pl.ds(v * _L, _L)] = idx + 2 * _NB
        bi0[pl.ds(v * _L, _L)] = idx + 3 * _NB
        gh[pl.ds(v * _L, _L)] = jnp.zeros((_L,), jnp.int32)
        return c

    lax.fori_loop(0, _NBV, ib, jnp.int32(0))
    fidx[...] = lane

    @pl.when(sid < 4)
    def _():
        pltpu.sync_copy(gh, shacc.at[pl.ds(sid * _NB, _NB)])

    @pl.when(sid == 4)
    def _():
        finv[...] = jnp.zeros((_L,), jnp.float32)
        pltpu.sync_copy(finv, shaccf)

    phi = jnp.int32(0)
    kr = jnp.int32(_K)

    for p in (3, 2, 1, 0):
        shift = 8 * p

        def zb(v, c):
            hist[pl.ds(v * _L, _L)] = jnp.zeros((_L,), jnp.int32)
            return c

        lax.fori_loop(0, _NBV, zb, jnp.int32(0))

        if p == 3:
            def hb(i, c):
                for j in range(2):
                    off = i * (2 * _L) + j * _L
                    xb = xv[pl.ds(off, _L)]
                    ibits = plsc.bitcast(xb, jnp.int32)
                    ks = jnp.where(ibits >= 0, ibits, ibits ^ _M31)
                    u = ks ^ _MININT
                    ub[pl.ds(off, _L)] = u
                    f = _srl(u, 24) & 0xFF
                    plsc.addupdate_scatter(hist, [f], ones)
                return c

            xcopy.wait()
            lax.fori_loop(0, _NV // 4, hb, jnp.int32(0))
            xcopy2.wait()
            lax.fori_loop(_NV // 4, _NV // 2, hb, jnp.int32(0))
        else:
            phiv = jnp.full((_L,), phi, jnp.int32)

            def hb(i, c, _shift=shift, _phiv=phiv):
                for j in range(2):
                    off = i * (2 * _L) + j * _L
                    u = ub[pl.ds(off, _L)]
                    act = _srl(u, _shift + 8) == _phiv
                    f = _srl(u, _shift) & 0xFF
                    plsc.addupdate_scatter(hist, [f], ones, mask=act)
                return c

            lax.fori_loop(0, _NV // 2, hb, jnp.int32(0))

        bip = {3: bi3, 2: bi2, 1: bi1, 0: bi0}[p]
        if p == 3:
            plsc.subcore_barrier()
        pltpu.sync_copy(hist, shacc.at[bip], add=True)
        plsc.subcore_barrier()
        pltpu.sync_copy(shacc.at[pl.ds((3 - p) * _NB, _NB)], gh)

        bidx = lane * _L
        tvec = plsc.load_gather(gh, [bidx])
        for j in range(1, _L):
            tvec = tvec + plsc.load_gather(gh, [bidx + j])
        st = lax.rev(plsc.cumsum(lax.rev(tvec, (0,))), (0,))
        condv = st >= kr
        vstar = jnp.sum(jnp.where(condv, 1, 0)) - 1
        stv = jnp.min(jnp.where(condv, st, _M31))
        hv = gh[pl.ds(vstar * _L, _L)]
        tv_ = jnp.sum(hv)
        s2 = lax.rev(plsc.cumsum(lax.rev(hv, (0,))), (0,)) + (stv - tv_)
        cond2 = s2 >= kr
        c2 = jnp.sum(jnp.where(cond2, 1, 0))
        sel_in = jnp.sum(jnp.where(cond2, hv, 0))
        bstar = vstar * _L + c2 - 1
        kr = kr - stv + sel_in
        phi = lax.shift_left(phi, jnp.int32(8)) | bstar

    phis = phi ^ _MININT

    def sm(i, a):
        for j in range(4):
            off = i * (4 * _L) + j * _L
            u = ub[pl.ds(off, _L)]
            xb = xv[pl.ds(off, _L)]
            gt = (u ^ _MININT) > phis
            a = a + jnp.where(gt, xb, jnp.float32(0))
        return a

    acc = lax.fori_loop(0, _NV // 4, sm, jnp.zeros((_L,), jnp.float32))
    sloc = jnp.sum(acc)

    finv[...] = jnp.full((_L,), sloc, jnp.float32)
    pltpu.sync_copy(finv, shaccf.at[fidx], add=True)
    plsc.subcore_barrier()
    pltpu.sync_copy(shaccf, faccv)

    stot = faccv[...]

    pv = jnp.full((_L,), phi, jnp.int32)
    ksv = pv ^ _MININT
    bits = jnp.where(ksv >= 0, ksv, ksv ^ _M31)
    tv = plsc.bitcast(bits, jnp.float32)
    krf = jnp.full((_L,), kr, jnp.int32).astype(jnp.float32)
    res = (stot + krf * tv) * jnp.float32(1.0 / _K)

    @pl.when((cid == 0) & (sid == 0))
    def _():
        outv[...] = res
        pltpu.sync_copy(outv.at[pl.ds(0, 1)], out_hbm)


def _make_kernel():
    mesh = plsc.VectorSubcoreMesh(core_axis_name="c", subcore_axis_name="s",
                                  num_cores=1)
    return pl.kernel(
        _sc_body,
        out_type=jax.ShapeDtypeStruct((1,), jnp.float32),
        mesh=mesh,
        compiler_params=pltpu.CompilerParams(needs_layout_passes=False),
        scratch_types=[
            pltpu.VMEM((_CHUNK,), jnp.float32),
            pltpu.VMEM((_CHUNK,), jnp.int32),
            pltpu.VMEM((_NB,), jnp.int32),
            pltpu.VMEM((_NB,), jnp.int32),
            pltpu.VMEM((_NB,), jnp.int32),
            pltpu.VMEM((_NB,), jnp.int32),
            pltpu.VMEM((_NB,), jnp.int32),
            pltpu.VMEM((_NB,), jnp.int32),
            pltpu.VMEM((_L,), jnp.int32),
            pltpu.VMEM((_L,), jnp.float32),
            pltpu.VMEM((_L,), jnp.float32),
            pltpu.VMEM((_L,), jnp.float32),
            pltpu.VMEM_SHARED((_FOFF,), jnp.int32),
            pltpu.VMEM_SHARED((_L,), jnp.float32),
            pltpu.SemaphoreType.DMA,
            pltpu.SemaphoreType.DMA,
        ],
    )


def kernel(x):
    out = _make_kernel()(x)
    return out.reshape(())
